# Initial kernel scaffold; baseline (speedup 1.0000x reference)
import jax
import jax.numpy as jnp
from jax.experimental import pallas as pl


def kernel(x, edge_index, x_node_id, y_node_id, arg_node_id, W_msg, W_cls, b_cls):
    raise NotImplementedError("write your pallas kernel here")


# speedup vs baseline: 5.8735x; 5.8735x over previous
import functools

import jax
import jax.numpy as jnp
from jax import lax
from jax.experimental import pallas as pl
from jax.experimental.pallas import tpu as pltpu
from jax.experimental.pallas import tpu_sc as plsc

N = 10000
D = 128
E = 320000
B = 64
K = 8
NC = 36

NWORKERS = 32
NSUB = 16
EPW = E // NWORKERS
CH = 80
NCHUNK = EPW // CH
NPAD = 10240
RPT = NPAD // NSUB
DR = NPAD // D

NIDX = 1536
IPW = NIDX // NWORKERS

_mesh = plsc.VectorSubcoreMesh(core_axis_name="c", subcore_axis_name="s")


@functools.partial(
    pl.kernel,
    out_type=(jax.ShapeDtypeStruct((NPAD, D), jnp.float32),
              jax.ShapeDtypeStruct((NPAD, D), jnp.float32),
              jax.ShapeDtypeStruct((DR, D), jnp.float32),
              jax.ShapeDtypeStruct((DR, D), jnp.float32)),
    mesh=_mesh,
    scratch_types=[
        pltpu.VMEM_SHARED((NPAD, D), jnp.float32),
        pltpu.VMEM_SHARED((DR, D), jnp.float32),
        pltpu.VMEM((CH,), jnp.int32),
        pltpu.VMEM((CH,), jnp.int32),
        pltpu.VMEM((CH, D), jnp.float32),
        pltpu.VMEM((DR, D), jnp.float32),
        pltpu.VMEM((DR,), jnp.int32),
        pltpu.SemaphoreType.DMA,
    ],
    compiler_params=pltpu.CompilerParams(needs_layout_passes=False),
)
def _sc_scatter(x_hbm, src_hbm, dst_hbm, s0_hbm, s1_hbm, d0_hbm, d1_hbm,
                acc, degacc, srcv, dstv, rowsv, hist, identv, sem):
    c = lax.axis_index("c")
    s = lax.axis_index("s")
    wid = c * NSUB + s
    row0 = pl.multiple_of(s * RPT, 8)
    zero16 = jnp.zeros((16,), jnp.float32)

    def zrow(r, _):
        for cc in range(D // 16):
            hist[r, pl.ds(cc * 16, 16)] = zero16
        return 0

    lax.fori_loop(0, DR, zrow, 0)
    for j in range(DR // 16):
        identv[pl.ds(j * 16, 16)] = lax.iota(jnp.int32, 16) + (j * 16)
    for j in range(RPT // DR):
        pltpu.sync_copy(hist, acc.at[pl.ds(row0 + j * DR, DR)])

    @pl.when(s == 0)
    def _():
        pltpu.sync_copy(hist, degacc)

    plsc.subcore_barrier()

    base = wid * EPW

    def body(i, _):
        off = pl.multiple_of(base + i * CH, 8)
        pltpu.sync_copy(src_hbm.at[pl.ds(off, CH)], srcv)
        pltpu.sync_copy(dst_hbm.at[pl.ds(off, CH)], dstv)
        pltpu.async_copy(x_hbm.at[srcv], rowsv, sem).wait()
        pltpu.sync_copy(rowsv, acc.at[dstv], add=True)
        for g in range(CH // 16):
            dv = dstv[pl.ds(g * 16, 16)]
            cnt, last = plsc.scan_count(dv)
            plsc.addupdate_scatter(
                hist,
                [lax.shift_right_logical(dv, 7), lax.bitwise_and(dv, 127)],
                cnt.astype(jnp.float32),
                mask=last,
            )
        return 0

    lax.fori_loop(0, NCHUNK, body, 0)
    plsc.subcore_barrier()
    pltpu.sync_copy(hist, degacc.at[identv], add=True)
    plsc.subcore_barrier()

    @pl.when(c == 0)
    def _():
        pltpu.sync_copy(acc.at[pl.ds(row0, RPT)], s0_hbm.at[pl.ds(row0, RPT)])

        @pl.when(s == 0)
        def _():
            pltpu.sync_copy(degacc, d0_hbm)

    @pl.when(c == 1)
    def _():
        pltpu.sync_copy(acc.at[pl.ds(row0, RPT)], s1_hbm.at[pl.ds(row0, RPT)])

        @pl.when(s == 0)
        def _():
            pltpu.sync_copy(degacc, d1_hbm)


@functools.partial(
    pl.kernel,
    out_type=(jax.ShapeDtypeStruct((NIDX, D), jnp.float32),
              jax.ShapeDtypeStruct((NIDX, D), jnp.float32),
              jax.ShapeDtypeStruct((NIDX, D), jnp.float32),
              jax.ShapeDtypeStruct((NIDX,), jnp.float32)),
    mesh=_mesh,
    scratch_types=[
        pltpu.VMEM((IPW,), jnp.int32),
        pltpu.VMEM((IPW, D), jnp.float32),
        pltpu.VMEM((IPW, D), jnp.float32),
        pltpu.VMEM((IPW, D), jnp.float32),
        pltpu.VMEM((DR, D), jnp.float32),
        pltpu.VMEM((DR, D), jnp.float32),
        pltpu.VMEM((IPW,), jnp.float32),
        pltpu.SemaphoreType.DMA,
    ],
    compiler_params=pltpu.CompilerParams(needs_layout_passes=False),
)
def _sc_gather(s0_hbm, s1_hbm, x_hbm, d0_hbm, d1_hbm, idx_hbm,
               o0, o1, ox, odeg, idxv, b0, b1, bx, d0v, d1v, degb, sem):
    c = lax.axis_index("c")
    s = lax.axis_index("s")
    base = pl.multiple_of((c * NSUB + s) * IPW, 8)
    pltpu.sync_copy(idx_hbm.at[pl.ds(base, IPW)], idxv)
    pltpu.sync_copy(d0_hbm, d0v)
    pltpu.sync_copy(d1_hbm, d1v)
    pltpu.async_copy(s0_hbm.at[idxv], b0, sem).wait()
    pltpu.async_copy(s1_hbm.at[idxv], b1, sem).wait()
    pltpu.async_copy(x_hbm.at[idxv], bx, sem).wait()
    for g in range(IPW // 16):
        dv = idxv[pl.ds(g * 16, 16)]
        r = lax.shift_right_logical(dv, 7)
        cc = lax.bitwise_and(dv, 127)
        degb[pl.ds(g * 16, 16)] = (plsc.load_gather(d0v, [r, cc])
                                   + plsc.load_gather(d1v, [r, cc]))
    pltpu.sync_copy(b0, o0.at[pl.ds(base, IPW)])
    pltpu.sync_copy(b1, o1.at[pl.ds(base, IPW)])
    pltpu.sync_copy(bx, ox.at[pl.ds(base, IPW)])
    pltpu.sync_copy(degb, odeg.at[pl.ds(base, IPW)])


def _tc_body(r0_ref, r1_ref, rx_ref, dg_ref, wm_ref, wc_ref, b_ref, out_ref):
    sm = r0_ref[...] + r1_ref[...]
    deg = dg_ref[...]
    agg = jnp.dot(sm, wm_ref[...], preferred_element_type=jnp.float32)
    h = jnp.maximum(rx_ref[...] + agg / jnp.maximum(deg, 1.0), 0.0)

    def pool(off):
        m = h[off:off + B]
        for j in range(1, K):
            m = jnp.maximum(m, h[off + j * B:off + (j + 1) * B])
        return m

    arga = pool(0)
    argb = pool(B * K)
    tempa = h[2 * B * K:2 * B * K + B]
    tempb = h[2 * B * K + B:2 * B * K + 2 * B]
    y = jnp.concatenate([arga, tempa, argb, tempb], axis=1)
    out_ref[...] = (
        jnp.dot(y, wc_ref[...], preferred_element_type=jnp.float32)
        + b_ref[...]
    )


_tc_final = pl.pallas_call(
    _tc_body,
    out_shape=jax.ShapeDtypeStruct((B, NC), jnp.float32),
)


def kernel(x, edge_index, x_node_id, y_node_id, arg_node_id,
           W_msg, W_cls, b_cls):
    src = edge_index[0].astype(jnp.int32)
    dst = edge_index[1].astype(jnp.int32)
    idx = jnp.concatenate([
        x_node_id.T.reshape(-1),
        y_node_id.T.reshape(-1),
        arg_node_id.T.reshape(-1),
        jnp.zeros((NIDX - 2 * B * K - 2 * B,), x_node_id.dtype),
    ]).astype(jnp.int32)

    s0, s1, d0, d1 = _sc_scatter(x, src, dst)
    r0, r1, rx, dr = _sc_gather(s0, s1, x, d0, d1, idx)
    return _tc_final(r0, r1, rx, dr.reshape(NIDX, 1),
                     W_msg, W_cls, b_cls.reshape(1, NC))


# speedup vs baseline: 11.0268x; 1.8774x over previous
import functools

import jax
import jax.numpy as jnp
from jax import lax
from jax.experimental import pallas as pl
from jax.experimental.pallas import tpu as pltpu
from jax.experimental.pallas import tpu_sc as plsc

N = 10000
D = 128
E = 320000
B = 64
K = 8
NC = 36

NWORKERS = 32
NSUB = 16
CH = 128
NCHI = 80
EPT = NCHI * CH
EPAD = NWORKERS * EPT
NBUF = 3
DEADNODE = 10000
NPAD = 10240

NIDX = 1536
IPW = NIDX // NWORKERS
DEADSLOT = NIDX
ACCROWS = 1664
SRT = ACCROWS // NSUB
DEGROWS = 16

_mesh = plsc.VectorSubcoreMesh(core_axis_name="c", subcore_axis_name="s")


@functools.partial(
    pl.kernel,
    out_type=(jax.ShapeDtypeStruct((ACCROWS, D), jnp.float32),
              jax.ShapeDtypeStruct((ACCROWS, D), jnp.float32),
              jax.ShapeDtypeStruct((DEGROWS, D), jnp.float32),
              jax.ShapeDtypeStruct((DEGROWS, D), jnp.float32),
              jax.ShapeDtypeStruct((NIDX,), jnp.int32)),
    mesh=_mesh,
    scratch_types=[
        pltpu.VMEM_SHARED((ACCROWS, D), jnp.float32),
        pltpu.VMEM_SHARED((DEGROWS, D), jnp.float32),
        pltpu.VMEM((NCHI, CH), jnp.int32),
        pltpu.VMEM((NCHI, CH), jnp.int32),
        pltpu.VMEM((NIDX,), jnp.int32),
        pltpu.VMEM((NPAD,), jnp.int32),
        pltpu.VMEM((EPT,), jnp.int32),
        pltpu.VMEM((EPT,), jnp.int32),
        pltpu.VMEM((NCHI, CH), jnp.int32),
        pltpu.VMEM((DEGROWS, D), jnp.float32),
        pltpu.VMEM((DEGROWS,), jnp.int32),
        pltpu.VMEM((IPW,), jnp.int32),
        [pltpu.VMEM((CH, D), jnp.float32) for _ in range(NBUF)],
        [pltpu.SemaphoreType.DMA for _ in range(NBUF)],
        pltpu.SemaphoreType.DMA,
        pltpu.SemaphoreType.DMA,
        pltpu.SemaphoreType.DMA,
        pltpu.SemaphoreType.DMA,
    ],
    compiler_params=pltpu.CompilerParams(needs_layout_passes=False),
)
def _sc_scatter(x_hbm, src_hbm, dst_hbm, idx_hbm,
                s0_hbm, s1_hbm, d0_hbm, d1_hbm, slots_hbm,
                acc, degacc, srcall, dstall, idxall, remap, csrcf, cslotf,
                cslot2d, hist, identv, slotl, rowsv, gsem, ssem,
                isem1, isem2, isem3):
    c = lax.axis_index("c")
    s = lax.axis_index("s")
    wid = c * NSUB + s
    zero16 = jnp.zeros((16,), jnp.float32)
    iota16 = lax.iota(jnp.int32, 16)

    pltpu.async_copy(src_hbm.at[wid], srcall, isem1)
    pltpu.async_copy(dst_hbm.at[wid], dstall, isem2)
    pltpu.async_copy(idx_hbm, idxall, isem3)

    for r in range(DEGROWS):
        for cc in range(D // 16):
            hist[r, pl.ds(cc * 16, 16)] = zero16
    identv[...] = iota16
    row0 = pl.multiple_of(s * SRT, 8)
    for j in range(SRT // DEGROWS):
        pltpu.sync_copy(hist, acc.at[pl.ds(row0 + j * DEGROWS, DEGROWS)])
    pltpu.sync_copy(hist.at[pl.ds(0, 8)],
                    acc.at[pl.ds(row0 + (SRT // DEGROWS) * DEGROWS, 8)])

    @pl.when(s == 0)
    def _():
        pltpu.sync_copy(hist, degacc)

    dead16 = jnp.full((16,), DEADSLOT, jnp.int32)

    def rinit(r, _):
        plsc.store_scatter(remap, [iota16 + r * 16], dead16)
        return 0

    lax.fori_loop(0, NPAD // 16, rinit, 0)

    zero16i = jnp.zeros((16,), jnp.int32)

    def cinit(r, _):
        pos = iota16 + r * 16
        plsc.store_scatter(csrcf, [pos], zero16i)
        plsc.store_scatter(cslotf, [pos], dead16)
        return 0

    lax.fori_loop(0, EPT // 16, cinit, 0)

    pltpu.make_async_copy(idx_hbm, idxall, isem3).wait()
    for q in range(NIDX // 16):
        dv = idxall[pl.ds(q * 16, 16)]
        _, last = plsc.scan_count(dv)
        plsc.store_scatter(remap, [dv], iota16 + q * 16, mask=last)

    pltpu.make_async_copy(src_hbm.at[wid], srcall, isem1).wait()
    pltpu.make_async_copy(dst_hbm.at[wid], dstall, isem2).wait()

    def p1(ci, off):
        for gg in range(CH // 16):
            sv = srcall[ci, pl.ds(gg * 16, 16)]
            dv = dstall[ci, pl.ds(gg * 16, 16)]
            slot = plsc.load_gather(remap, [dv])
            m = slot != DEADSLOT
            mi = m.astype(jnp.int32)
            pos = off - 1 + plsc.cumsum(mi)
            plsc.store_scatter(csrcf, [pos], sv, mask=m)
            plsc.store_scatter(cslotf, [pos], slot, mask=m)
            off = off + jnp.sum(mi)
        return off

    m_kept = lax.fori_loop(0, NCHI, p1, jnp.int32(0))

    def p15(r, _):
        sl = plsc.load_gather(cslotf, [iota16 + r * 16])
        plsc.store_scatter(
            cslot2d,
            [jnp.full((16,), r >> 3, jnp.int32) >> 0,
             iota16 + (lax.bitwise_and(r, 7) * 16)],
            sl)
        return 0

    lax.fori_loop(0, EPT // 16, p15, 0)

    plsc.subcore_barrier()

    nch = lax.shift_right_logical(m_kept + (CH - 1), 7)
    trip = jnp.maximum(lax.div(nch + (NBUF - 1), NBUF), 1)
    totch = trip * NBUF

    for b in range(NBUF):
        off_b = pl.multiple_of(b * CH, CH)
        pltpu.async_copy(
            x_hbm.at[csrcf.at[pl.ds(off_b, CH)]], rowsv[b], gsem[b])

    def ring(g, _):
        for b in range(NBUF):
            i = g * NBUF + b
            offi = pl.multiple_of(i * CH, CH)
            pltpu.make_async_copy(
                x_hbm.at[csrcf.at[pl.ds(offi, CH)]], rowsv[b],
                gsem[b]).wait()
            sc = pltpu.async_copy(
                rowsv[b], acc.at[cslot2d.at[i]], ssem, add=True)
            for gg in range(CH // 16):
                sl = plsc.load_gather(cslotf, [iota16 + (offi + gg * 16)])
                cnt, last = plsc.scan_count(sl)
                plsc.addupdate_scatter(
                    hist,
                    [lax.shift_right_logical(sl, 7),
                     lax.bitwise_and(sl, 127)],
                    cnt.astype(jnp.float32),
                    mask=last,
                )
            sc.wait()
            nxt = i + NBUF

            @pl.when(nxt < totch)
            def _():
                offn = pl.multiple_of(nxt * CH, CH)
                pltpu.async_copy(
                    x_hbm.at[csrcf.at[pl.ds(offn, CH)]], rowsv[b], gsem[b])
        return 0

    lax.fori_loop(0, trip, ring, 0)
    plsc.subcore_barrier()
    pltpu.sync_copy(hist, degacc.at[identv], add=True)
    plsc.subcore_barrier()

    @pl.when(c == 0)
    def _():
        pltpu.sync_copy(acc.at[pl.ds(row0, SRT)], s0_hbm.at[pl.ds(row0, SRT)])

        @pl.when(s == 0)
        def _():
            pltpu.sync_copy(degacc, d0_hbm)

    @pl.when(c == 1)
    def _():
        pltpu.sync_copy(acc.at[pl.ds(row0, SRT)], s1_hbm.at[pl.ds(row0, SRT)])

        @pl.when(s == 0)
        def _():
            pltpu.sync_copy(degacc, d1_hbm)

    base = pl.multiple_of(wid * IPW, 8)
    for g in range(IPW // 16):
        dv = plsc.load_gather(idxall, [iota16 + (base + g * 16)])
        slotl[pl.ds(g * 16, 16)] = plsc.load_gather(remap, [dv])
    pltpu.sync_copy(slotl, slots_hbm.at[pl.ds(base, IPW)])


@functools.partial(
    pl.kernel,
    out_type=(jax.ShapeDtypeStruct((NIDX, D), jnp.float32),
              jax.ShapeDtypeStruct((NIDX, D), jnp.float32),
              jax.ShapeDtypeStruct((NIDX, D), jnp.float32),
              jax.ShapeDtypeStruct((NIDX,), jnp.float32)),
    mesh=_mesh,
    scratch_types=[
        pltpu.VMEM((IPW,), jnp.int32),
        pltpu.VMEM((IPW,), jnp.int32),
        pltpu.VMEM((IPW, D), jnp.float32),
        pltpu.VMEM((IPW, D), jnp.float32),
        pltpu.VMEM((IPW, D), jnp.float32),
        pltpu.VMEM((DEGROWS, D), jnp.float32),
        pltpu.VMEM((DEGROWS, D), jnp.float32),
        pltpu.VMEM((IPW,), jnp.float32),
        pltpu.SemaphoreType.DMA,
    ],
    compiler_params=pltpu.CompilerParams(needs_layout_passes=False),
)
def _sc_gather(s0_hbm, s1_hbm, x_hbm, d0_hbm, d1_hbm, idx_hbm, slots_hbm,
               o0, o1, ox, odeg,
               idxv, slotv, b0, b1, bx, d0v, d1v, degb, sem):
    c = lax.axis_index("c")
    s = lax.axis_index("s")
    base = pl.multiple_of((c * NSUB + s) * IPW, 8)
    iota16 = lax.iota(jnp.int32, 16)
    pltpu.sync_copy(idx_hbm.at[pl.ds(base, IPW)], idxv)
    pltpu.sync_copy(slots_hbm.at[pl.ds(base, IPW)], slotv)
    pltpu.sync_copy(d0_hbm, d0v)
    pltpu.sync_copy(d1_hbm, d1v)
    pltpu.async_copy(s0_hbm.at[slotv], b0, sem).wait()
    pltpu.async_copy(s1_hbm.at[slotv], b1, sem).wait()
    pltpu.async_copy(x_hbm.at[idxv], bx, sem).wait()
    for g in range(IPW // 16):
        sl = slotv[pl.ds(g * 16, 16)]
        r = lax.shift_right_logical(sl, 7)
        cc = lax.bitwise_and(sl, 127)
        degb[pl.ds(g * 16, 16)] = (plsc.load_gather(d0v, [r, cc])
                                   + plsc.load_gather(d1v, [r, cc]))
    pltpu.sync_copy(b0, o0.at[pl.ds(base, IPW)])
    pltpu.sync_copy(b1, o1.at[pl.ds(base, IPW)])
    pltpu.sync_copy(bx, ox.at[pl.ds(base, IPW)])
    pltpu.sync_copy(degb, odeg.at[pl.ds(base, IPW)])


def _tc_body(r0_ref, r1_ref, rx_ref, dg_ref, wm_ref, wc_ref, b_ref, out_ref):
    sm = r0_ref[...] + r1_ref[...]
    deg = dg_ref[...]
    agg = jnp.dot(sm, wm_ref[...], preferred_element_type=jnp.float32)
    h = jnp.maximum(rx_ref[...] + agg / jnp.maximum(deg, 1.0), 0.0)

    def pool(off):
        m = h[off:off + B]
        for j in range(1, K):
            m = jnp.maximum(m, h[off + j * B:off + (j + 1) * B])
        return m

    arga = pool(0)
    argb = pool(B * K)
    tempa = h[2 * B * K:2 * B * K + B]
    tempb = h[2 * B * K + B:2 * B * K + 2 * B]
    y = jnp.concatenate([arga, tempa, argb, tempb], axis=1)
    out_ref[...] = (
        jnp.dot(y, wc_ref[...], preferred_element_type=jnp.float32)
        + b_ref[...]
    )


_tc_final = pl.pallas_call(
    _tc_body,
    out_shape=jax.ShapeDtypeStruct((B, NC), jnp.float32),
)


def kernel(x, edge_index, x_node_id, y_node_id, arg_node_id,
           W_msg, W_cls, b_cls):
    src = jnp.concatenate([
        edge_index[0].astype(jnp.int32),
        jnp.zeros((EPAD - E,), jnp.int32),
    ]).reshape(NWORKERS, NCHI, CH)
    dst = jnp.concatenate([
        edge_index[1].astype(jnp.int32),
        jnp.full((EPAD - E,), DEADNODE, jnp.int32),
    ]).reshape(NWORKERS, NCHI, CH)
    idx = jnp.concatenate([
        x_node_id.T.reshape(-1),
        y_node_id.T.reshape(-1),
        arg_node_id.T.reshape(-1),
        jnp.zeros((NIDX - 2 * B * K - 2 * B,), x_node_id.dtype),
    ]).astype(jnp.int32)

    s0, s1, d0, d1, slots = _sc_scatter(x, src, dst, idx)
    r0, r1, rx, dr = _sc_gather(s0, s1, x, d0, d1, idx, slots)
    return _tc_final(r0, r1, rx, dr.reshape(NIDX, 1),
                     W_msg, W_cls, b_cls.reshape(1, NC))


# speedup vs baseline: 12.0118x; 1.0893x over previous
import functools

import jax
import jax.numpy as jnp
from jax import lax
from jax.experimental import pallas as pl
from jax.experimental.pallas import tpu as pltpu
from jax.experimental.pallas import tpu_sc as plsc

N = 10000
D = 128
E = 320000
B = 64
K = 8
NC = 36

NWORKERS = 32
NSUB = 16
CH = 128
NCHI = 80
EPT = NCHI * CH
EPAD = NWORKERS * EPT
NBUF = 3
DEADNODE = 10000
NPAD = 10240

NIDX = 1536
IPW = NIDX // NWORKERS
DEADSLOT = NIDX
ACCROWS = 1664
SRT = ACCROWS // NSUB
DEGROWS = 16

_mesh = plsc.VectorSubcoreMesh(core_axis_name="c", subcore_axis_name="s")


@functools.partial(
    pl.kernel,
    out_type=(jax.ShapeDtypeStruct((ACCROWS, D), jnp.float32),
              jax.ShapeDtypeStruct((ACCROWS, D), jnp.float32),
              jax.ShapeDtypeStruct((DEGROWS, D), jnp.float32),
              jax.ShapeDtypeStruct((DEGROWS, D), jnp.float32),
              jax.ShapeDtypeStruct((NIDX,), jnp.int32)),
    mesh=_mesh,
    scratch_types=[
        pltpu.VMEM_SHARED((ACCROWS, D), jnp.float32),
        pltpu.VMEM_SHARED((DEGROWS, D), jnp.float32),
        pltpu.VMEM((NCHI, CH), jnp.int32),
        pltpu.VMEM((NCHI, CH), jnp.int32),
        pltpu.VMEM((NIDX,), jnp.int32),
        pltpu.VMEM((NPAD,), jnp.int32),
        pltpu.VMEM((NCHI, CH), jnp.int32),
        pltpu.VMEM((NCHI, CH), jnp.int32),
        pltpu.VMEM((DEGROWS, D), jnp.float32),
        pltpu.VMEM((DEGROWS,), jnp.int32),
        pltpu.VMEM((IPW,), jnp.int32),
        [pltpu.VMEM((CH, D), jnp.float32) for _ in range(NBUF)],
        [pltpu.SemaphoreType.DMA for _ in range(NBUF)],
        pltpu.SemaphoreType.DMA,
        pltpu.SemaphoreType.DMA,
        pltpu.SemaphoreType.DMA,
        pltpu.SemaphoreType.DMA,
        pltpu.SemaphoreType.DMA,
    ],
    compiler_params=pltpu.CompilerParams(needs_layout_passes=False),
)
def _sc_scatter(x_hbm, src_hbm, dst_hbm, idx_hbm, ri_hbm,
                s0_hbm, s1_hbm, d0_hbm, d1_hbm, slots_hbm,
                acc, degacc, srcall, dstall, idxall, remap,
                csrc2d, cslot2d, hist, identv, slotl, rowsv, gsem, ssem,
                isem1, isem2, isem3, isem4):
    c = lax.axis_index("c")
    s = lax.axis_index("s")
    wid = c * NSUB + s
    zero16 = jnp.zeros((16,), jnp.float32)
    iota16 = lax.iota(jnp.int32, 16)

    pltpu.async_copy(src_hbm.at[wid], srcall, isem1)
    pltpu.async_copy(dst_hbm.at[wid], dstall, isem2)
    pltpu.async_copy(idx_hbm, idxall, isem3)
    pltpu.async_copy(ri_hbm, remap, isem4)

    for r in range(DEGROWS):
        for cc in range(D // 16):
            hist[r, pl.ds(cc * 16, 16)] = zero16
    identv[...] = iota16
    row0 = pl.multiple_of(s * SRT, 8)
    for j in range(SRT // DEGROWS):
        pltpu.sync_copy(hist, acc.at[pl.ds(row0 + j * DEGROWS, DEGROWS)])
    pltpu.sync_copy(hist.at[pl.ds(0, 8)],
                    acc.at[pl.ds(row0 + (SRT // DEGROWS) * DEGROWS, 8)])

    @pl.when(s == 0)
    def _():
        pltpu.sync_copy(hist, degacc)

    dead16 = jnp.full((16,), DEADSLOT, jnp.int32)
    zero16i = jnp.zeros((16,), jnp.int32)

    pltpu.make_async_copy(idx_hbm, idxall, isem3).wait()
    pltpu.make_async_copy(ri_hbm, remap, isem4).wait()
    for q in range(NIDX // 16):
        dv = idxall[pl.ds(q * 16, 16)]
        _, last = plsc.scan_count(dv)
        plsc.store_scatter(remap, [dv], iota16 + q * 16, mask=last)

    pltpu.make_async_copy(src_hbm.at[wid], srcall, isem1).wait()
    pltpu.make_async_copy(dst_hbm.at[wid], dstall, isem2).wait()

    def p1(ci, off):
        for gg in range(CH // 16):
            sv = srcall[ci, pl.ds(gg * 16, 16)]
            dv = dstall[ci, pl.ds(gg * 16, 16)]
            slot = plsc.load_gather(remap, [dv])
            m = slot != DEADSLOT
            mi = m.astype(jnp.int32)
            pos = off - 1 + plsc.cumsum(mi)
            pr = lax.shift_right_logical(pos, 7)
            pc = lax.bitwise_and(pos, 127)
            plsc.store_scatter(csrc2d, [pr, pc], sv, mask=m)
            plsc.store_scatter(cslot2d, [pr, pc], slot, mask=m)
            off = off + jnp.sum(mi)
        return off

    m_kept = lax.fori_loop(0, NCHI, p1, jnp.int32(0))

    for k in range(NBUF * CH // 16):
        pos = m_kept + iota16 + k * 16
        mok = pos < EPT
        pr = lax.shift_right_logical(pos, 7)
        pc = lax.bitwise_and(pos, 127)
        plsc.store_scatter(csrc2d, [pr, pc], zero16i, mask=mok)
        plsc.store_scatter(cslot2d, [pr, pc], dead16, mask=mok)

    plsc.subcore_barrier()

    nch = lax.shift_right_logical(m_kept + (CH - 1), 7)
    trip = jnp.maximum(lax.div(nch + (NBUF - 1), NBUF), 1)
    totch = trip * NBUF

    for b in range(NBUF):
        pltpu.async_copy(x_hbm.at[csrc2d.at[b]], rowsv[b], gsem[b])

    def ring(g, _):
        for b in range(NBUF):
            i = g * NBUF + b
            pltpu.make_async_copy(
                x_hbm.at[csrc2d.at[i]], rowsv[b], gsem[b]).wait()
            sc = pltpu.async_copy(
                rowsv[b], acc.at[cslot2d.at[i]], ssem, add=True)
            for gg in range(CH // 16):
                sl = cslot2d[i, pl.ds(gg * 16, 16)]
                cnt, last = plsc.scan_count(sl)
                plsc.addupdate_scatter(
                    hist,
                    [lax.shift_right_logical(sl, 7),
                     lax.bitwise_and(sl, 127)],
                    cnt.astype(jnp.float32),
                    mask=last,
                )
            sc.wait()
            nxt = i + NBUF

            @pl.when(nxt < totch)
            def _():
                pltpu.async_copy(x_hbm.at[csrc2d.at[nxt]], rowsv[b], gsem[b])
        return 0

    lax.fori_loop(0, trip, ring, 0)
    plsc.subcore_barrier()
    pltpu.sync_copy(hist, degacc.at[identv], add=True)
    plsc.subcore_barrier()

    @pl.when(c == 0)
    def _():
        pltpu.sync_copy(acc.at[pl.ds(row0, SRT)], s0_hbm.at[pl.ds(row0, SRT)])

        @pl.when(s == 0)
        def _():
            pltpu.sync_copy(degacc, d0_hbm)

    @pl.when(c == 1)
    def _():
        pltpu.sync_copy(acc.at[pl.ds(row0, SRT)], s1_hbm.at[pl.ds(row0, SRT)])

        @pl.when(s == 0)
        def _():
            pltpu.sync_copy(degacc, d1_hbm)

    base = pl.multiple_of(wid * IPW, 8)
    for g in range(IPW // 16):
        dv = plsc.load_gather(idxall, [iota16 + (base + g * 16)])
        slotl[pl.ds(g * 16, 16)] = plsc.load_gather(remap, [dv])
    pltpu.sync_copy(slotl, slots_hbm.at[pl.ds(base, IPW)])


@functools.partial(
    pl.kernel,
    out_type=(jax.ShapeDtypeStruct((NIDX, D), jnp.float32),
              jax.ShapeDtypeStruct((NIDX, D), jnp.float32),
              jax.ShapeDtypeStruct((NIDX, D), jnp.float32),
              jax.ShapeDtypeStruct((NIDX,), jnp.float32)),
    mesh=_mesh,
    scratch_types=[
        pltpu.VMEM((IPW,), jnp.int32),
        pltpu.VMEM((IPW,), jnp.int32),
        pltpu.VMEM((IPW, D), jnp.float32),
        pltpu.VMEM((IPW, D), jnp.float32),
        pltpu.VMEM((IPW, D), jnp.float32),
        pltpu.VMEM((DEGROWS, D), jnp.float32),
        pltpu.VMEM((DEGROWS, D), jnp.float32),
        pltpu.VMEM((IPW,), jnp.float32),
        [pltpu.SemaphoreType.DMA for _ in range(11)],
    ],
    compiler_params=pltpu.CompilerParams(needs_layout_passes=False),
)
def _sc_gather(s0_hbm, s1_hbm, x_hbm, d0_hbm, d1_hbm, idx_hbm, slots_hbm,
               o0, o1, ox, odeg,
               idxv, slotv, b0, b1, bx, d0v, d1v, degb, sems):
    c = lax.axis_index("c")
    s = lax.axis_index("s")
    base = pl.multiple_of((c * NSUB + s) * IPW, 8)
    pltpu.async_copy(idx_hbm.at[pl.ds(base, IPW)], idxv, sems[0])
    pltpu.async_copy(slots_hbm.at[pl.ds(base, IPW)], slotv, sems[1])
    pltpu.async_copy(d0_hbm, d0v, sems[2])
    pltpu.async_copy(d1_hbm, d1v, sems[3])
    pltpu.make_async_copy(slots_hbm.at[pl.ds(base, IPW)], slotv,
                          sems[1]).wait()
    pltpu.async_copy(s0_hbm.at[slotv], b0, sems[4])
    pltpu.async_copy(s1_hbm.at[slotv], b1, sems[5])
    pltpu.make_async_copy(idx_hbm.at[pl.ds(base, IPW)], idxv, sems[0]).wait()
    pltpu.async_copy(x_hbm.at[idxv], bx, sems[6])
    pltpu.make_async_copy(d0_hbm, d0v, sems[2]).wait()
    pltpu.make_async_copy(d1_hbm, d1v, sems[3]).wait()
    for g in range(IPW // 16):
        sl = slotv[pl.ds(g * 16, 16)]
        r = lax.shift_right_logical(sl, 7)
        cc = lax.bitwise_and(sl, 127)
        degb[pl.ds(g * 16, 16)] = (plsc.load_gather(d0v, [r, cc])
                                   + plsc.load_gather(d1v, [r, cc]))
    pltpu.async_copy(degb, odeg.at[pl.ds(base, IPW)], sems[10])
    pltpu.make_async_copy(s0_hbm.at[slotv], b0, sems[4]).wait()
    pltpu.async_copy(b0, o0.at[pl.ds(base, IPW)], sems[7])
    pltpu.make_async_copy(s1_hbm.at[slotv], b1, sems[5]).wait()
    pltpu.async_copy(b1, o1.at[pl.ds(base, IPW)], sems[8])
    pltpu.make_async_copy(x_hbm.at[idxv], bx, sems[6]).wait()
    pltpu.async_copy(bx, ox.at[pl.ds(base, IPW)], sems[9])
    pltpu.make_async_copy(degb, odeg.at[pl.ds(base, IPW)], sems[10]).wait()
    pltpu.make_async_copy(b0, o0.at[pl.ds(base, IPW)], sems[7]).wait()
    pltpu.make_async_copy(b1, o1.at[pl.ds(base, IPW)], sems[8]).wait()
    pltpu.make_async_copy(bx, ox.at[pl.ds(base, IPW)], sems[9]).wait()


def _tc_body(r0_ref, r1_ref, rx_ref, dg_ref, wm_ref, wc_ref, b_ref, out_ref):
    sm = r0_ref[...] + r1_ref[...]
    deg = dg_ref[...]
    agg = jnp.dot(sm, wm_ref[...], preferred_element_type=jnp.float32)
    h = jnp.maximum(rx_ref[...] + agg / jnp.maximum(deg, 1.0), 0.0)

    def pool(off):
        m = h[off:off + B]
        for j in range(1, K):
            m = jnp.maximum(m, h[off + j * B:off + (j + 1) * B])
        return m

    arga = pool(0)
    argb = pool(B * K)
    tempa = h[2 * B * K:2 * B * K + B]
    tempb = h[2 * B * K + B:2 * B * K + 2 * B]
    y = jnp.concatenate([arga, tempa, argb, tempb], axis=1)
    out_ref[...] = (
        jnp.dot(y, wc_ref[...], preferred_element_type=jnp.float32)
        + b_ref[...]
    )


_tc_final = pl.pallas_call(
    _tc_body,
    out_shape=jax.ShapeDtypeStruct((B, NC), jnp.float32),
)


def kernel(x, edge_index, x_node_id, y_node_id, arg_node_id,
           W_msg, W_cls, b_cls):
    src = jnp.concatenate([
        edge_index[0].astype(jnp.int32),
        jnp.zeros((EPAD - E,), jnp.int32),
    ]).reshape(NWORKERS, NCHI, CH)
    dst = jnp.concatenate([
        edge_index[1].astype(jnp.int32),
        jnp.full((EPAD - E,), DEADNODE, jnp.int32),
    ]).reshape(NWORKERS, NCHI, CH)
    idx = jnp.concatenate([
        x_node_id.T.reshape(-1),
        y_node_id.T.reshape(-1),
        arg_node_id.T.reshape(-1),
        jnp.zeros((NIDX - 2 * B * K - 2 * B,), x_node_id.dtype),
    ]).astype(jnp.int32)
    ri = jnp.full((NPAD,), DEADSLOT, jnp.int32)

    s0, s1, d0, d1, slots = _sc_scatter(x, src, dst, idx, ri)
    r0, r1, rx, dr = _sc_gather(s0, s1, x, d0, d1, idx, slots)
    return _tc_final(r0, r1, rx, dr.reshape(NIDX, 1),
                     W_msg, W_cls, b_cls.reshape(1, NC))


# speedup vs baseline: 12.5854x; 1.0478x over previous
import functools

import jax
import jax.numpy as jnp
from jax import lax
from jax.experimental import pallas as pl
from jax.experimental.pallas import tpu as pltpu
from jax.experimental.pallas import tpu_sc as plsc

N = 10000
D = 128
E = 320000
B = 64
K = 8
NC = 36

NWORKERS = 32
NSUB = 16
CH = 128
NCHI = 80
EPT = NCHI * CH
EPAD = NWORKERS * EPT
NBUF = 3
DEADNODE = 10000
NPAD = 10240

NIDX = 1536
IPW = NIDX // NWORKERS
DEADSLOT = NIDX
ACCROWS = 1664
SRT = ACCROWS // NSUB
DEGROWS = 16

_mesh = plsc.VectorSubcoreMesh(core_axis_name="c", subcore_axis_name="s")


@functools.partial(
    pl.kernel,
    out_type=(jax.ShapeDtypeStruct((ACCROWS, D), jnp.float32),
              jax.ShapeDtypeStruct((ACCROWS, D), jnp.float32),
              jax.ShapeDtypeStruct((DEGROWS, D), jnp.float32),
              jax.ShapeDtypeStruct((DEGROWS, D), jnp.float32),
              jax.ShapeDtypeStruct((NIDX,), jnp.int32)),
    mesh=_mesh,
    scratch_types=[
        pltpu.VMEM_SHARED((ACCROWS, D), jnp.float32),
        pltpu.VMEM_SHARED((DEGROWS, D), jnp.float32),
        pltpu.VMEM((NCHI, CH), jnp.int32),
        pltpu.VMEM((NCHI, CH), jnp.int32),
        pltpu.VMEM((NIDX,), jnp.int32),
        pltpu.VMEM((NPAD,), jnp.int32),
        pltpu.VMEM((NCHI, CH), jnp.int32),
        pltpu.VMEM((NCHI, CH), jnp.int32),
        pltpu.VMEM((DEGROWS, D), jnp.float32),
        pltpu.VMEM((DEGROWS,), jnp.int32),
        pltpu.VMEM((IPW,), jnp.int32),
        [pltpu.VMEM((CH, D), jnp.float32) for _ in range(NBUF)],
        [pltpu.SemaphoreType.DMA for _ in range(NBUF)],
        pltpu.SemaphoreType.DMA,
        pltpu.SemaphoreType.DMA,
        pltpu.SemaphoreType.DMA,
        pltpu.SemaphoreType.DMA,
        pltpu.SemaphoreType.DMA,
    ],
    compiler_params=pltpu.CompilerParams(needs_layout_passes=False),
)
def _sc_scatter(x_hbm, src_hbm, dst_hbm, idx_hbm, ri_hbm,
                s0_hbm, s1_hbm, d0_hbm, d1_hbm, slots_hbm,
                acc, degacc, srcall, dstall, idxall, remap,
                csrc2d, cslot2d, hist, identv, slotl, rowsv, gsem, ssem,
                isem1, isem2, isem3, isem4):
    c = lax.axis_index("c")
    s = lax.axis_index("s")
    wid = c * NSUB + s
    zero16 = jnp.zeros((16,), jnp.float32)
    iota16 = lax.iota(jnp.int32, 16)

    pltpu.async_copy(src_hbm.at[wid], srcall, isem1)
    pltpu.async_copy(dst_hbm.at[wid], dstall, isem2)
    pltpu.async_copy(idx_hbm, idxall, isem3)
    pltpu.async_copy(ri_hbm, remap, isem4)

    for r in range(DEGROWS):
        for cc in range(D // 16):
            hist[r, pl.ds(cc * 16, 16)] = zero16
    identv[...] = iota16
    row0 = pl.multiple_of(s * SRT, 8)
    zsem = gsem[0]
    for j in range(SRT // DEGROWS):
        pltpu.async_copy(hist, acc.at[pl.ds(row0 + j * DEGROWS, DEGROWS)],
                         zsem)
    pltpu.async_copy(hist.at[pl.ds(0, 8)],
                     acc.at[pl.ds(row0 + (SRT // DEGROWS) * DEGROWS, 8)],
                     zsem)

    @pl.when(s == 0)
    def _():
        pltpu.sync_copy(hist, degacc)

    for j in range(SRT // DEGROWS):
        pltpu.make_async_copy(
            hist, acc.at[pl.ds(row0 + j * DEGROWS, DEGROWS)], zsem).wait()
    pltpu.make_async_copy(
        hist.at[pl.ds(0, 8)],
        acc.at[pl.ds(row0 + (SRT // DEGROWS) * DEGROWS, 8)], zsem).wait()

    dead16 = jnp.full((16,), DEADSLOT, jnp.int32)
    zero16i = jnp.zeros((16,), jnp.int32)

    pltpu.make_async_copy(idx_hbm, idxall, isem3).wait()
    pltpu.make_async_copy(ri_hbm, remap, isem4).wait()
    for q in range(NIDX // 16):
        dv = idxall[pl.ds(q * 16, 16)]
        _, last = plsc.scan_count(dv)
        plsc.store_scatter(remap, [dv], iota16 + q * 16, mask=last)

    pltpu.make_async_copy(src_hbm.at[wid], srcall, isem1).wait()
    pltpu.make_async_copy(dst_hbm.at[wid], dstall, isem2).wait()

    def p1(ci, off):
        svs, slots, masks, cums = [], [], [], []
        for gg in range(CH // 16):
            sv = srcall[ci, pl.ds(gg * 16, 16)]
            dv = dstall[ci, pl.ds(gg * 16, 16)]
            slot = plsc.load_gather(remap, [dv])
            m = slot != DEADSLOT
            svs.append(sv)
            slots.append(slot)
            masks.append(m)
            cums.append(plsc.cumsum(m.astype(jnp.int32)))
        offs = [off]
        for gg in range(CH // 16):
            offs.append(offs[gg] + cums[gg][15])
        for gg in range(CH // 16):
            pos = offs[gg] - 1 + cums[gg]
            pr = lax.shift_right_logical(pos, 7)
            pc = lax.bitwise_and(pos, 127)
            plsc.store_scatter(csrc2d, [pr, pc], svs[gg], mask=masks[gg])
            plsc.store_scatter(cslot2d, [pr, pc], slots[gg], mask=masks[gg])
        return offs[CH // 16]

    m_kept = lax.fori_loop(0, NCHI, p1, jnp.int32(0))

    for k in range(NBUF * CH // 16):
        pos = m_kept + iota16 + k * 16
        mok = pos < EPT
        pr = lax.shift_right_logical(pos, 7)
        pc = lax.bitwise_and(pos, 127)
        plsc.store_scatter(csrc2d, [pr, pc], zero16i, mask=mok)
        plsc.store_scatter(cslot2d, [pr, pc], dead16, mask=mok)

    plsc.subcore_barrier()

    nch = lax.shift_right_logical(m_kept + (CH - 1), 7)
    trip = jnp.maximum(lax.div(nch + (NBUF - 1), NBUF), 1)
    totch = trip * NBUF

    for b in range(NBUF):
        pltpu.async_copy(x_hbm.at[csrc2d.at[b]], rowsv[b], gsem[b])

    def ring(g, _):
        for b in range(NBUF):
            i = g * NBUF + b
            pltpu.make_async_copy(
                x_hbm.at[csrc2d.at[i]], rowsv[b], gsem[b]).wait()
            sc = pltpu.async_copy(
                rowsv[b], acc.at[cslot2d.at[i]], ssem, add=True)
            for gg in range(CH // 16):
                sl = cslot2d[i, pl.ds(gg * 16, 16)]
                cnt, last = plsc.scan_count(sl)
                plsc.addupdate_scatter(
                    hist,
                    [lax.shift_right_logical(sl, 7),
                     lax.bitwise_and(sl, 127)],
                    cnt.astype(jnp.float32),
                    mask=last,
                )
            sc.wait()
            nxt = i + NBUF

            @pl.when(nxt < totch)
            def _():
                pltpu.async_copy(x_hbm.at[csrc2d.at[nxt]], rowsv[b], gsem[b])
        return 0

    lax.fori_loop(0, trip, ring, 0)
    plsc.subcore_barrier()
    pltpu.sync_copy(hist, degacc.at[identv], add=True)
    plsc.subcore_barrier()

    @pl.when(c == 0)
    def _():
        pltpu.sync_copy(acc.at[pl.ds(row0, SRT)], s0_hbm.at[pl.ds(row0, SRT)])

        @pl.when(s == 0)
        def _():
            pltpu.sync_copy(degacc, d0_hbm)

    @pl.when(c == 1)
    def _():
        pltpu.sync_copy(acc.at[pl.ds(row0, SRT)], s1_hbm.at[pl.ds(row0, SRT)])

        @pl.when(s == 0)
        def _():
            pltpu.sync_copy(degacc, d1_hbm)

    base = pl.multiple_of(wid * IPW, 8)
    for g in range(IPW // 16):
        dv = plsc.load_gather(idxall, [iota16 + (base + g * 16)])
        slotl[pl.ds(g * 16, 16)] = plsc.load_gather(remap, [dv])
    pltpu.sync_copy(slotl, slots_hbm.at[pl.ds(base, IPW)])


@functools.partial(
    pl.kernel,
    out_type=(jax.ShapeDtypeStruct((NIDX, D), jnp.float32),
              jax.ShapeDtypeStruct((NIDX, D), jnp.float32),
              jax.ShapeDtypeStruct((NIDX, D), jnp.float32),
              jax.ShapeDtypeStruct((NIDX,), jnp.float32)),
    mesh=_mesh,
    scratch_types=[
        pltpu.VMEM((IPW,), jnp.int32),
        pltpu.VMEM((IPW,), jnp.int32),
        pltpu.VMEM((IPW, D), jnp.float32),
        pltpu.VMEM((IPW, D), jnp.float32),
        pltpu.VMEM((IPW, D), jnp.float32),
        pltpu.VMEM((DEGROWS, D), jnp.float32),
        pltpu.VMEM((DEGROWS, D), jnp.float32),
        pltpu.VMEM((IPW,), jnp.float32),
        [pltpu.SemaphoreType.DMA for _ in range(11)],
    ],
    compiler_params=pltpu.CompilerParams(needs_layout_passes=False),
)
def _sc_gather(s0_hbm, s1_hbm, x_hbm, d0_hbm, d1_hbm, idx_hbm, slots_hbm,
               o0, o1, ox, odeg,
               idxv, slotv, b0, b1, bx, d0v, d1v, degb, sems):
    c = lax.axis_index("c")
    s = lax.axis_index("s")
    base = pl.multiple_of((c * NSUB + s) * IPW, 8)
    pltpu.async_copy(idx_hbm.at[pl.ds(base, IPW)], idxv, sems[0])
    pltpu.async_copy(slots_hbm.at[pl.ds(base, IPW)], slotv, sems[1])
    pltpu.async_copy(d0_hbm, d0v, sems[2])
    pltpu.async_copy(d1_hbm, d1v, sems[3])
    pltpu.make_async_copy(slots_hbm.at[pl.ds(base, IPW)], slotv,
                          sems[1]).wait()
    pltpu.async_copy(s0_hbm.at[slotv], b0, sems[4])
    pltpu.async_copy(s1_hbm.at[slotv], b1, sems[5])
    pltpu.make_async_copy(idx_hbm.at[pl.ds(base, IPW)], idxv, sems[0]).wait()
    pltpu.async_copy(x_hbm.at[idxv], bx, sems[6])
    pltpu.make_async_copy(d0_hbm, d0v, sems[2]).wait()
    pltpu.make_async_copy(d1_hbm, d1v, sems[3]).wait()
    for g in range(IPW // 16):
        sl = slotv[pl.ds(g * 16, 16)]
        r = lax.shift_right_logical(sl, 7)
        cc = lax.bitwise_and(sl, 127)
        degb[pl.ds(g * 16, 16)] = (plsc.load_gather(d0v, [r, cc])
                                   + plsc.load_gather(d1v, [r, cc]))
    pltpu.async_copy(degb, odeg.at[pl.ds(base, IPW)], sems[10])
    pltpu.make_async_copy(s0_hbm.at[slotv], b0, sems[4]).wait()
    pltpu.async_copy(b0, o0.at[pl.ds(base, IPW)], sems[7])
    pltpu.make_async_copy(s1_hbm.at[slotv], b1, sems[5]).wait()
    pltpu.async_copy(b1, o1.at[pl.ds(base, IPW)], sems[8])
    pltpu.make_async_copy(x_hbm.at[idxv], bx, sems[6]).wait()
    pltpu.async_copy(bx, ox.at[pl.ds(base, IPW)], sems[9])
    pltpu.make_async_copy(degb, odeg.at[pl.ds(base, IPW)], sems[10]).wait()
    pltpu.make_async_copy(b0, o0.at[pl.ds(base, IPW)], sems[7]).wait()
    pltpu.make_async_copy(b1, o1.at[pl.ds(base, IPW)], sems[8]).wait()
    pltpu.make_async_copy(bx, ox.at[pl.ds(base, IPW)], sems[9]).wait()


def _tc_body(r0_ref, r1_ref, rx_ref, dg_ref, wm_ref, wc_ref, b_ref, out_ref):
    sm = r0_ref[...] + r1_ref[...]
    deg = dg_ref[...]
    agg = jnp.dot(sm, wm_ref[...], preferred_element_type=jnp.float32)
    h = jnp.maximum(rx_ref[...] + agg / jnp.maximum(deg, 1.0), 0.0)

    def pool(off):
        m = h[off:off + B]
        for j in range(1, K):
            m = jnp.maximum(m, h[off + j * B:off + (j + 1) * B])
        return m

    arga = pool(0)
    argb = pool(B * K)
    tempa = h[2 * B * K:2 * B * K + B]
    tempb = h[2 * B * K + B:2 * B * K + 2 * B]
    y = jnp.concatenate([arga, tempa, argb, tempb], axis=1)
    out_ref[...] = (
        jnp.dot(y, wc_ref[...], preferred_element_type=jnp.float32)
        + b_ref[...]
    )


_tc_final = pl.pallas_call(
    _tc_body,
    out_shape=jax.ShapeDtypeStruct((B, NC), jnp.float32),
)


def kernel(x, edge_index, x_node_id, y_node_id, arg_node_id,
           W_msg, W_cls, b_cls):
    src = jnp.concatenate([
        edge_index[0].astype(jnp.int32),
        jnp.zeros((EPAD - E,), jnp.int32),
    ]).reshape(NWORKERS, NCHI, CH)
    dst = jnp.concatenate([
        edge_index[1].astype(jnp.int32),
        jnp.full((EPAD - E,), DEADNODE, jnp.int32),
    ]).reshape(NWORKERS, NCHI, CH)
    idx = jnp.concatenate([
        x_node_id.T.reshape(-1),
        y_node_id.T.reshape(-1),
        arg_node_id.T.reshape(-1),
        jnp.zeros((NIDX - 2 * B * K - 2 * B,), x_node_id.dtype),
    ]).astype(jnp.int32)
    ri = jnp.full((NPAD,), DEADSLOT, jnp.int32)

    s0, s1, d0, d1, slots = _sc_scatter(x, src, dst, idx, ri)
    r0, r1, rx, dr = _sc_gather(s0, s1, x, d0, d1, idx, slots)
    return _tc_final(r0, r1, rx, dr.reshape(NIDX, 1),
                     W_msg, W_cls, b_cls.reshape(1, NC))


# speedup vs baseline: 12.6197x; 1.0027x over previous
import functools

import jax
import jax.numpy as jnp
from jax import lax
from jax.experimental import pallas as pl
from jax.experimental.pallas import tpu as pltpu
from jax.experimental.pallas import tpu_sc as plsc

N = 10000
D = 128
E = 320000
B = 64
K = 8
NC = 36

NWORKERS = 32
NSUB = 16
CH = 128
NCHI = 80
EPT = NCHI * CH
EPAD = NWORKERS * EPT
NBUF = 3
DEADNODE = 10000
NPAD = 10240

NIDX = 1536
IPW = NIDX // NWORKERS
DEADSLOT = NIDX
ACCROWS = 1664
SRT = ACCROWS // NSUB
DEGROWS = 16

_mesh = plsc.VectorSubcoreMesh(core_axis_name="c", subcore_axis_name="s")


@functools.partial(
    pl.kernel,
    out_type=(jax.ShapeDtypeStruct((ACCROWS, D), jnp.float32),
              jax.ShapeDtypeStruct((ACCROWS, D), jnp.float32),
              jax.ShapeDtypeStruct((DEGROWS, D), jnp.float32),
              jax.ShapeDtypeStruct((DEGROWS, D), jnp.float32),
              jax.ShapeDtypeStruct((NIDX,), jnp.int32)),
    mesh=_mesh,
    scratch_types=[
        pltpu.VMEM_SHARED((ACCROWS, D), jnp.float32),
        pltpu.VMEM_SHARED((DEGROWS, D), jnp.float32),
        pltpu.VMEM((NCHI, CH), jnp.int32),
        pltpu.VMEM((NCHI, CH), jnp.int32),
        pltpu.VMEM((NIDX,), jnp.int32),
        pltpu.VMEM((NPAD,), jnp.int32),
        pltpu.VMEM((NCHI, CH), jnp.int32),
        pltpu.VMEM((NCHI, CH), jnp.int32),
        pltpu.VMEM((DEGROWS, D), jnp.float32),
        pltpu.VMEM((DEGROWS,), jnp.int32),
        pltpu.VMEM((IPW,), jnp.int32),
        [pltpu.VMEM((CH, D), jnp.float32) for _ in range(NBUF)],
        [pltpu.SemaphoreType.DMA for _ in range(NBUF)],
        pltpu.SemaphoreType.DMA,
        pltpu.SemaphoreType.DMA,
        pltpu.SemaphoreType.DMA,
        pltpu.SemaphoreType.DMA,
        pltpu.SemaphoreType.DMA,
    ],
    compiler_params=pltpu.CompilerParams(needs_layout_passes=False),
)
def _sc_scatter(x_hbm, src_hbm, dst_hbm, idx_hbm, ri_hbm,
                s0_hbm, s1_hbm, d0_hbm, d1_hbm, slots_hbm,
                acc, degacc, srcall, dstall, idxall, remap,
                csrc2d, cslot2d, hist, identv, slotl, rowsv, gsem, ssem,
                isem1, isem2, isem3, isem4):
    c = lax.axis_index("c")
    s = lax.axis_index("s")
    wid = c * NSUB + s
    zero16 = jnp.zeros((16,), jnp.float32)
    iota16 = lax.iota(jnp.int32, 16)

    pltpu.async_copy(src_hbm.at[wid], srcall, isem1)
    pltpu.async_copy(dst_hbm.at[wid], dstall, isem2)
    pltpu.async_copy(idx_hbm, idxall, isem3)
    pltpu.async_copy(ri_hbm, remap, isem4)

    for r in range(DEGROWS):
        for cc in range(D // 16):
            hist[r, pl.ds(cc * 16, 16)] = zero16
    identv[...] = iota16
    row0 = pl.multiple_of(s * SRT, 8)
    zsem = gsem[0]
    for j in range(SRT // DEGROWS):
        pltpu.async_copy(hist, acc.at[pl.ds(row0 + j * DEGROWS, DEGROWS)],
                         zsem)
    pltpu.async_copy(hist.at[pl.ds(0, 8)],
                     acc.at[pl.ds(row0 + (SRT // DEGROWS) * DEGROWS, 8)],
                     zsem)

    @pl.when(s == 0)
    def _():
        pltpu.sync_copy(hist, degacc)

    for j in range(SRT // DEGROWS):
        pltpu.make_async_copy(
            hist, acc.at[pl.ds(row0 + j * DEGROWS, DEGROWS)], zsem).wait()
    pltpu.make_async_copy(
        hist.at[pl.ds(0, 8)],
        acc.at[pl.ds(row0 + (SRT // DEGROWS) * DEGROWS, 8)], zsem).wait()

    dead16 = jnp.full((16,), DEADSLOT, jnp.int32)
    zero16i = jnp.zeros((16,), jnp.int32)

    pltpu.make_async_copy(idx_hbm, idxall, isem3).wait()
    pltpu.make_async_copy(ri_hbm, remap, isem4).wait()
    for q in range(NIDX // 16):
        dv = idxall[pl.ds(q * 16, 16)]
        _, last = plsc.scan_count(dv)
        plsc.store_scatter(remap, [dv], iota16 + q * 16, mask=last)

    with jax.named_scope("wait_edges"):
        pltpu.make_async_copy(src_hbm.at[wid], srcall, isem1).wait()
        pltpu.make_async_copy(dst_hbm.at[wid], dstall, isem2).wait()

    def p1(ci, off):
        svs, slots, masks, cums = [], [], [], []
        for gg in range(CH // 16):
            sv = srcall[ci, pl.ds(gg * 16, 16)]
            dv = dstall[ci, pl.ds(gg * 16, 16)]
            slot = plsc.load_gather(remap, [dv])
            m = slot != DEADSLOT
            svs.append(sv)
            slots.append(slot)
            masks.append(m)
            cums.append(plsc.cumsum(m.astype(jnp.int32)))
        offs = [off]
        for gg in range(CH // 16):
            offs.append(offs[gg] + cums[gg][15])
        for gg in range(CH // 16):
            pos = offs[gg] - 1 + cums[gg]
            pr = lax.shift_right_logical(pos, 7)
            pc = lax.bitwise_and(pos, 127)
            plsc.store_scatter(csrc2d, [pr, pc], svs[gg], mask=masks[gg])
            plsc.store_scatter(cslot2d, [pr, pc], slots[gg], mask=masks[gg])
        return offs[CH // 16]

    with jax.named_scope("p1_filter"):
        m_kept = lax.fori_loop(0, NCHI, p1, jnp.int32(0))

    for k in range(NBUF * CH // 16):
        pos = m_kept + iota16 + k * 16
        mok = pos < EPT
        pr = lax.shift_right_logical(pos, 7)
        pc = lax.bitwise_and(pos, 127)
        plsc.store_scatter(csrc2d, [pr, pc], zero16i, mask=mok)
        plsc.store_scatter(cslot2d, [pr, pc], dead16, mask=mok)

    plsc.subcore_barrier()

    nch = lax.shift_right_logical(m_kept + (CH - 1), 7)
    trip = jnp.maximum(lax.div(nch + (NBUF - 1), NBUF), 1)
    totch = trip * NBUF

    for b in range(NBUF):
        pltpu.async_copy(x_hbm.at[csrc2d.at[b]], rowsv[b], gsem[b])

    def ring(g, _):
        for b in range(NBUF):
            i = g * NBUF + b
            pltpu.make_async_copy(
                x_hbm.at[csrc2d.at[i]], rowsv[b], gsem[b]).wait()
            sc = pltpu.async_copy(
                rowsv[b], acc.at[cslot2d.at[i]], ssem, add=True)
            for gg in range(CH // 16):
                sl = cslot2d[i, pl.ds(gg * 16, 16)]
                cnt, last = plsc.scan_count(sl)
                plsc.addupdate_scatter(
                    hist,
                    [lax.shift_right_logical(sl, 7),
                     lax.bitwise_and(sl, 127)],
                    cnt.astype(jnp.float32),
                    mask=last,
                )
            sc.wait()
            nxt = i + NBUF

            @pl.when(nxt < totch)
            def _():
                pltpu.async_copy(x_hbm.at[csrc2d.at[nxt]], rowsv[b], gsem[b])
        return 0

    with jax.named_scope("ring"):
        lax.fori_loop(0, trip, ring, 0)
    with jax.named_scope("epilogue"):
        plsc.subcore_barrier()
        pltpu.sync_copy(hist, degacc.at[identv], add=True)
        plsc.subcore_barrier()

    @pl.when(c == 0)
    def _():
        pltpu.sync_copy(acc.at[pl.ds(row0, SRT)], s0_hbm.at[pl.ds(row0, SRT)])

        @pl.when(s == 0)
        def _():
            pltpu.sync_copy(degacc, d0_hbm)

    @pl.when(c == 1)
    def _():
        pltpu.sync_copy(acc.at[pl.ds(row0, SRT)], s1_hbm.at[pl.ds(row0, SRT)])

        @pl.when(s == 0)
        def _():
            pltpu.sync_copy(degacc, d1_hbm)

    base = pl.multiple_of(wid * IPW, 8)
    for g in range(IPW // 16):
        dv = plsc.load_gather(idxall, [iota16 + (base + g * 16)])
        slotl[pl.ds(g * 16, 16)] = plsc.load_gather(remap, [dv])
    pltpu.sync_copy(slotl, slots_hbm.at[pl.ds(base, IPW)])


@functools.partial(
    pl.kernel,
    out_type=(jax.ShapeDtypeStruct((NIDX, D), jnp.float32),
              jax.ShapeDtypeStruct((NIDX, D), jnp.float32),
              jax.ShapeDtypeStruct((NIDX, D), jnp.float32),
              jax.ShapeDtypeStruct((NIDX,), jnp.float32)),
    mesh=_mesh,
    scratch_types=[
        pltpu.VMEM((IPW,), jnp.int32),
        pltpu.VMEM((IPW,), jnp.int32),
        pltpu.VMEM((IPW, D), jnp.float32),
        pltpu.VMEM((IPW, D), jnp.float32),
        pltpu.VMEM((IPW, D), jnp.float32),
        pltpu.VMEM((DEGROWS, D), jnp.float32),
        pltpu.VMEM((DEGROWS, D), jnp.float32),
        pltpu.VMEM((IPW,), jnp.float32),
        [pltpu.SemaphoreType.DMA for _ in range(11)],
    ],
    compiler_params=pltpu.CompilerParams(needs_layout_passes=False),
)
def _sc_gather(s0_hbm, s1_hbm, x_hbm, d0_hbm, d1_hbm, idx_hbm, slots_hbm,
               o0, o1, ox, odeg,
               idxv, slotv, b0, b1, bx, d0v, d1v, degb, sems):
    c = lax.axis_index("c")
    s = lax.axis_index("s")
    base = pl.multiple_of((c * NSUB + s) * IPW, 8)
    pltpu.async_copy(idx_hbm.at[pl.ds(base, IPW)], idxv, sems[0])
    pltpu.async_copy(slots_hbm.at[pl.ds(base, IPW)], slotv, sems[1])
    pltpu.async_copy(d0_hbm, d0v, sems[2])
    pltpu.async_copy(d1_hbm, d1v, sems[3])
    pltpu.make_async_copy(slots_hbm.at[pl.ds(base, IPW)], slotv,
                          sems[1]).wait()
    pltpu.async_copy(s0_hbm.at[slotv], b0, sems[4])
    pltpu.async_copy(s1_hbm.at[slotv], b1, sems[5])
    pltpu.make_async_copy(idx_hbm.at[pl.ds(base, IPW)], idxv, sems[0]).wait()
    pltpu.async_copy(x_hbm.at[idxv], bx, sems[6])
    pltpu.make_async_copy(d0_hbm, d0v, sems[2]).wait()
    pltpu.make_async_copy(d1_hbm, d1v, sems[3]).wait()
    for g in range(IPW // 16):
        sl = slotv[pl.ds(g * 16, 16)]
        r = lax.shift_right_logical(sl, 7)
        cc = lax.bitwise_and(sl, 127)
        degb[pl.ds(g * 16, 16)] = (plsc.load_gather(d0v, [r, cc])
                                   + plsc.load_gather(d1v, [r, cc]))
    pltpu.async_copy(degb, odeg.at[pl.ds(base, IPW)], sems[10])
    pltpu.make_async_copy(s0_hbm.at[slotv], b0, sems[4]).wait()
    pltpu.async_copy(b0, o0.at[pl.ds(base, IPW)], sems[7])
    pltpu.make_async_copy(s1_hbm.at[slotv], b1, sems[5]).wait()
    pltpu.async_copy(b1, o1.at[pl.ds(base, IPW)], sems[8])
    pltpu.make_async_copy(x_hbm.at[idxv], bx, sems[6]).wait()
    pltpu.async_copy(bx, ox.at[pl.ds(base, IPW)], sems[9])
    pltpu.make_async_copy(degb, odeg.at[pl.ds(base, IPW)], sems[10]).wait()
    pltpu.make_async_copy(b0, o0.at[pl.ds(base, IPW)], sems[7]).wait()
    pltpu.make_async_copy(b1, o1.at[pl.ds(base, IPW)], sems[8]).wait()
    pltpu.make_async_copy(bx, ox.at[pl.ds(base, IPW)], sems[9]).wait()


def _tc_body(r0_ref, r1_ref, rx_ref, dg_ref, wm_ref, wc_ref, b_ref, out_ref):
    sm = r0_ref[...] + r1_ref[...]
    deg = dg_ref[...]
    agg = jnp.dot(sm, wm_ref[...], preferred_element_type=jnp.float32)
    h = jnp.maximum(rx_ref[...] + agg / jnp.maximum(deg, 1.0), 0.0)

    def pool(off):
        m = h[off:off + B]
        for j in range(1, K):
            m = jnp.maximum(m, h[off + j * B:off + (j + 1) * B])
        return m

    arga = pool(0)
    argb = pool(B * K)
    tempa = h[2 * B * K:2 * B * K + B]
    tempb = h[2 * B * K + B:2 * B * K + 2 * B]
    y = jnp.concatenate([arga, tempa, argb, tempb], axis=1)
    out_ref[...] = (
        jnp.dot(y, wc_ref[...], preferred_element_type=jnp.float32)
        + b_ref[...]
    )


_tc_final = pl.pallas_call(
    _tc_body,
    out_shape=jax.ShapeDtypeStruct((B, NC), jnp.float32),
)


def kernel(x, edge_index, x_node_id, y_node_id, arg_node_id,
           W_msg, W_cls, b_cls):
    src = jnp.concatenate([
        edge_index[0].astype(jnp.int32),
        jnp.zeros((EPAD - E,), jnp.int32),
    ]).reshape(NWORKERS, NCHI, CH)
    dst = jnp.concatenate([
        edge_index[1].astype(jnp.int32),
        jnp.full((EPAD - E,), DEADNODE, jnp.int32),
    ]).reshape(NWORKERS, NCHI, CH)
    idx = jnp.concatenate([
        x_node_id.T.reshape(-1),
        y_node_id.T.reshape(-1),
        arg_node_id.T.reshape(-1),
        jnp.zeros((NIDX - 2 * B * K - 2 * B,), x_node_id.dtype),
    ]).astype(jnp.int32)
    ri = jnp.full((NPAD,), DEADSLOT, jnp.int32)

    s0, s1, d0, d1, slots = _sc_scatter(x, src, dst, idx, ri)
    r0, r1, rx, dr = _sc_gather(s0, s1, x, d0, d1, idx, slots)
    return _tc_final(r0, r1, rx, dr.reshape(NIDX, 1),
                     W_msg, W_cls, b_cls.reshape(1, NC))
